# baseline (speedup 1.0000x reference)
import math

import jax
import jax.numpy as jnp
from jax import lax
from jax.experimental import pallas as pl
from jax.experimental.pallas import tpu as pltpu

_NUM_CLASSES = 1000


def _round_up(a: int, b: int) -> int:
    return ((a + b - 1) // b) * b


def _head_kernel(x_ref, w1_ref, b1_ref, w2_ref, b2_ref, o_ref):
    xb = x_ref[...].astype(jnp.bfloat16)
    h = jnp.dot(xb, w1_ref[...], preferred_element_type=jnp.float32)
    a = jnp.maximum(h + b1_ref[...], 0.0).astype(jnp.bfloat16)
    out = lax.dot_general(w2_ref[...], a, (((0,), (1,)), ((), ())),
                          preferred_element_type=jnp.float32)
    out = out + b2_ref[...]
    o_ref[...] = out[: o_ref.shape[0], :].astype(o_ref.dtype)


def kernel(x, w1, b1, w2, b2):
    lead_shape = x.shape[:-1]
    rows = math.prod(lead_shape) if lead_shape else 1
    dh = w1.shape[0]
    nc = _NUM_CLASSES
    nc_w = w2.shape[1]

    tm = 2048
    rows_p = _round_up(rows, tm)

    x2d = x.reshape(rows, dh)
    if rows_p != rows:
        x2d = jnp.pad(x2d, ((0, rows_p - rows), (0, 0)))

    w1b = w1.astype(jnp.bfloat16)
    w2b = w2.astype(jnp.bfloat16)
    b1f = b1.astype(jnp.float32).reshape(1, dh)
    b2f = b2.astype(jnp.float32).reshape(nc_w, 1)

    footprint = (dh * dh * 2 + dh * nc_w * 2
                 + (dh + nc_w) * 4
                 + 2 * tm * dh * 4
                 + tm * dh * 4
                 + 2 * tm * nc_w * 4)

    cost = pl.CostEstimate(
        flops=2 * rows_p * dh * dh + 2 * rows_p * dh * nc_w,
        transcendentals=0,
        bytes_accessed=(rows_p * dh * 4 + dh * dh * 2 + dh * nc_w * 2
                        + (dh + nc_w) * 4 + rows_p * nc * 4),
    )

    out = pl.pallas_call(
        _head_kernel,
        out_shape=jax.ShapeDtypeStruct((nc, rows_p), x.dtype),
        grid=(rows_p // tm,),
        in_specs=[
            pl.BlockSpec((tm, dh), lambda i: (i, 0),
                         pipeline_mode=pl.Buffered(2)),
            pl.BlockSpec((dh, dh), lambda i: (0, 0),
                         pipeline_mode=pl.Buffered(1)),
            pl.BlockSpec((1, dh), lambda i: (0, 0),
                         pipeline_mode=pl.Buffered(1)),
            pl.BlockSpec((dh, nc_w), lambda i: (0, 0),
                         pipeline_mode=pl.Buffered(1)),
            pl.BlockSpec((nc_w, 1), lambda i: (0, 0),
                         pipeline_mode=pl.Buffered(1)),
        ],
        out_specs=pl.BlockSpec((nc, tm), lambda i: (0, i)),
        compiler_params=pltpu.CompilerParams(
            dimension_semantics=("arbitrary",),
            vmem_limit_bytes=int(min(footprint * 5 // 4 + (2 << 20), 100 << 20))),
        cost_estimate=cost,
    )(x2d, w1b, b1f, w2b, b2f)

    if rows_p != rows:
        out = out[:, :rows]
    return out.T.reshape(*lead_shape, nc)


# speedup vs baseline: 1.0580x; 1.0580x over previous
import math

import jax
import jax.numpy as jnp
from jax import lax
from jax.experimental import pallas as pl
from jax.experimental.pallas import tpu as pltpu

_NUM_CLASSES = 1000


def _round_up(a: int, b: int) -> int:
    return ((a + b - 1) // b) * b


def _head_kernel(x_ref, w1_ref, b1_ref, w2_ref, b2_ref, o_ref):
    xb = x_ref[...].astype(jnp.bfloat16)
    h = jnp.dot(xb, w1_ref[...], preferred_element_type=jnp.float32)
    a = jnp.maximum(h + b1_ref[...], 0.0).astype(jnp.bfloat16)
    out = lax.dot_general(w2_ref[...], a, (((0,), (1,)), ((), ())),
                          preferred_element_type=jnp.float32)
    out = out + b2_ref[...]
    o_ref[...] = out[: o_ref.shape[0], :].astype(o_ref.dtype)


def kernel(x, w1, b1, w2, b2):
    lead_shape = x.shape[:-1]
    rows = math.prod(lead_shape) if lead_shape else 1
    dh = w1.shape[0]
    nc = _NUM_CLASSES
    nc_w = w2.shape[1]

    tm = 1024
    rows_p = _round_up(rows, tm)

    x2d = x.reshape(rows, dh)
    if rows_p != rows:
        x2d = jnp.pad(x2d, ((0, rows_p - rows), (0, 0)))

    w1b = w1.astype(jnp.bfloat16)
    w2b = w2.astype(jnp.bfloat16)
    b1f = b1.astype(jnp.float32).reshape(1, dh)
    b2f = b2.astype(jnp.float32).reshape(nc_w, 1)

    footprint = (dh * dh * 2 + dh * nc_w * 2
                 + (dh + nc_w) * 4
                 + 2 * tm * dh * 4
                 + tm * dh * 4
                 + 2 * tm * nc_w * 4)

    cost = pl.CostEstimate(
        flops=2 * rows_p * dh * dh + 2 * rows_p * dh * nc_w,
        transcendentals=0,
        bytes_accessed=(rows_p * dh * 4 + dh * dh * 2 + dh * nc_w * 2
                        + (dh + nc_w) * 4 + rows_p * nc * 4),
    )

    out = pl.pallas_call(
        _head_kernel,
        out_shape=jax.ShapeDtypeStruct((nc, rows_p), x.dtype),
        grid=(rows_p // tm,),
        in_specs=[
            pl.BlockSpec((tm, dh), lambda i: (i, 0),
                         pipeline_mode=pl.Buffered(2)),
            pl.BlockSpec((dh, dh), lambda i: (0, 0),
                         pipeline_mode=pl.Buffered(1)),
            pl.BlockSpec((1, dh), lambda i: (0, 0),
                         pipeline_mode=pl.Buffered(1)),
            pl.BlockSpec((dh, nc_w), lambda i: (0, 0),
                         pipeline_mode=pl.Buffered(1)),
            pl.BlockSpec((nc_w, 1), lambda i: (0, 0),
                         pipeline_mode=pl.Buffered(1)),
        ],
        out_specs=pl.BlockSpec((nc, tm), lambda i: (0, i)),
        compiler_params=pltpu.CompilerParams(
            dimension_semantics=("arbitrary",),
            vmem_limit_bytes=int(min(footprint * 5 // 4 + (2 << 20), 100 << 20))),
        cost_estimate=cost,
    )(x2d, w1b, b1f, w2b, b2f)

    if rows_p != rows:
        out = out[:, :rows]
    return out.T.reshape(*lead_shape, nc)
